# baseline (speedup 1.0000x reference)
import jax
import jax.numpy as jnp
import numpy as np
from jax.experimental import pallas as pl
from jax.experimental.pallas import tpu as pltpu
from jax.experimental.pallas import tpu_sc as plsc

N = 10000
E = 160000
B = 64
CTX = 512
PED = 240
HID = 128
IN_DIM = HID + PED + CTX
IN_PAD = 896
OUT = 1024

NB = 1000
NGRID = N // NB

NSC = 2
NTEC = 16
EPT = E // (NSC * NTEC)
RC = 512
NCHUNK = 21
NP = NCHUNK * RC
RCA = 640
G = 32
NSCAN = (EPT + 15) // 16
HITCAP = 5120
NPL = IN_PAD // 128

_DT = np.exp(np.arange(0, PED // 2, dtype=np.float32) * -(np.log(10000.0) / (PED // 2)))
_CD = PED // 3
_INV_BN = float(1.0 / np.sqrt(1.0 + 1e-5))
_INV_SQRT2 = float(1.0 / np.sqrt(2.0))


def _prep_body(x_ref, pos_ref, batch_ref, ctx_ref, w0_ref, b0_ref, out_ref):
    xb = x_ref[0, 0, :]
    oh = (xb[:, None] == jax.lax.broadcasted_iota(jnp.int32, (NB, 118), 1)).astype(jnp.float32)
    h0 = jnp.dot(oh, w0_ref[...], preferred_element_type=jnp.float32) + b0_ref[...]
    pb = pos_ref[0]
    dt = jnp.exp(jax.lax.broadcasted_iota(jnp.int32, (1, _CD // 2), 1).astype(jnp.float32)
                 * (-(np.log(10000.0) / (PED // 2))))
    parts = []
    for i in range(3):
        s = pb[:, i:i + 1] * dt
        parts.append(jnp.concatenate([jnp.sin(s), jnp.cos(s)], axis=-1))
    pe = jnp.concatenate(parts, axis=1)
    bb = batch_ref[0, 0, :]
    ohb = (bb[:, None] == jax.lax.broadcasted_iota(jnp.int32, (NB, B), 1)).astype(jnp.float32)
    ctxg = jnp.dot(ohb, ctx_ref[...], preferred_element_type=jnp.float32)
    pad = jnp.zeros((NB, IN_PAD - IN_DIM), dtype=jnp.float32)
    xcb = jnp.concatenate([h0, pe, ctxg, pad], axis=1)
    for k in range(NPL):
        out_ref[:, k, :] = xcb[:, k * 128:(k + 1) * 128]


def _prep(x2, pos3, batch2, ctx, W0, b0):
    return pl.pallas_call(
        _prep_body,
        grid=(NGRID,),
        in_specs=[
            pl.BlockSpec((1, 1, NB), lambda i: (i, 0, 0)),
            pl.BlockSpec((1, NB, 3), lambda i: (i, 0, 0)),
            pl.BlockSpec((1, 1, NB), lambda i: (i, 0, 0)),
            pl.BlockSpec((B, CTX), lambda i: (0, 0)),
            pl.BlockSpec((118, HID), lambda i: (0, 0)),
            pl.BlockSpec((HID,), lambda i: (0,)),
        ],
        out_specs=pl.BlockSpec((NB, NPL, 128), lambda i: (i, 0, 0)),
        out_shape=jax.ShapeDtypeStruct((NP, NPL, 128), jnp.float32),
    )(x2, pos3, batch2, ctx, W0, b0)


EB = 2000


def _emb_body(ea_ref, we_ref, be_ref, out_ref):
    emb = jnp.dot(ea_ref[...], we_ref[...], preferred_element_type=jnp.float32) + be_ref[...]
    for k in range(NPL):
        out_ref[:, k, :] = emb[:, k * 128:(k + 1) * 128]


def _emb(edge_attr, We_pad, be_pad):
    return pl.pallas_call(
        _emb_body,
        grid=(E // EB,),
        in_specs=[
            pl.BlockSpec((EB, 5), lambda i: (i, 0)),
            pl.BlockSpec((5, IN_PAD), lambda i: (0, 0)),
            pl.BlockSpec((IN_PAD,), lambda i: (0,)),
        ],
        out_specs=pl.BlockSpec((EB, NPL, 128), lambda i: (i, 0, 0)),
        out_shape=jax.ShapeDtypeStruct((E, NPL, 128), jnp.float32),
    )(edge_attr, We_pad, be_pad)


def _edge_body(xc_hbm, src_hbm, dst_hbm, emb_hbm, z_hbm,
               out_hbm,
               src_v, dst_v, srch, dsth, ehh,
               sgA, sgB, egA, egB, dgA, dgB, erowsA, erowsB, rowsA, rowsB,
               acc, sem, sem2, sem3):
    cid = jax.lax.axis_index("c")
    sid = jax.lax.axis_index("s")
    wid = cid * NTEC + sid
    base = wid * EPT
    pltpu.sync_copy(src_hbm.at[pl.ds(base, EPT)], src_v.at[pl.ds(0, EPT)])
    pltpu.sync_copy(dst_hbm.at[pl.ds(base, EPT)], dst_v.at[pl.ds(0, EPT)])
    lanes = jax.lax.iota(jnp.int32, 16)
    full = lanes >= 0

    def chunk_body(c, carry):
        lo = c * RC
        pltpu.sync_copy(z_hbm, acc.at[pl.ds(sid * (RCA * NPL // NTEC), RCA * NPL // NTEC)])
        plsc.subcore_barrier()

        def scan_body(i, ptr):
            d = dst_v[pl.ds(i * 16, 16)]
            s = src_v[pl.ds(i * 16, 16)]
            eidx = i * 16 + lanes
            m = (d >= lo) & (d < lo + RC) & (eidx < EPT)
            mi = m.astype(jnp.int32)
            pos = ptr + plsc.cumsum(mi) - 1
            plsc.store_scatter(srch, [pos], s, mask=m)
            plsc.store_scatter(dsth, [pos], d - lo, mask=m)
            plsc.store_scatter(ehh, [pos], base + eidx, mask=m)
            return ptr + jnp.sum(mi)

        nh = jax.lax.fori_loop(0, NSCAN, scan_body, jnp.int32(0))
        zeros16 = jnp.zeros((16,), jnp.int32)
        for off in (0, 16):
            pos = nh + off + lanes
            plsc.store_scatter(srch, [pos], zeros16, mask=full)
            plsc.store_scatter(dsth, [pos], zeros16 + RC, mask=full)
            plsc.store_scatter(ehh, [pos], zeros16, mask=full)
        ng = (nh + (G - 1)) // G

        def group_body(g, _):
            gb = g * G
            rowsb = (rowsA, rowsB)
            erowsb = (erowsA, erowsB)
            sgb = (sgA, sgB)
            egb = (egA, egB)
            dgb = (dgA, dgB)
            gcop = []
            for h in range(2):
                s16 = srch[pl.ds(gb + h * 16, 16)]
                e16 = ehh[pl.ds(gb + h * 16, 16)]
                d16 = dsth[pl.ds(gb + h * 16, 16)]
                for k in range(NPL):
                    pos = lanes * NPL + k
                    plsc.store_scatter(sgb[h], [pos], s16 * NPL + k)
                    plsc.store_scatter(egb[h], [pos], e16 * NPL + k)
                    plsc.store_scatter(dgb[h], [pos], d16 * NPL + k)
                gcop.append([pltpu.async_copy(xc_hbm.at[sgb[h]], rowsb[h], sem),
                             pltpu.async_copy(emb_hbm.at[egb[h]], erowsb[h], sem2)])

            scat = []
            for h in range(2):
                for cpy in gcop[h]:
                    cpy.wait()
                rw, er = rowsb[h], erowsb[h]

                def r_body(r, _, rw=rw, er=er):
                    for jj in range(8):
                        cs = pl.ds(jj * 16, 16)
                        v = rw[r, cs] + er[r, cs]
                        rw[r, cs] = jnp.maximum(v, 0.0)
                    return 0

                jax.lax.fori_loop(0, 16 * NPL, r_body, 0)
                scat.append(pltpu.async_copy(rw, acc.at[dgb[h]], sem3, add=True))
            for s_ in scat:
                s_.wait()
            return 0

        jax.lax.fori_loop(0, ng, group_body, 0)
        plsc.subcore_barrier()
        pltpu.sync_copy(acc.at[pl.ds(sid * (RC * NPL // NTEC), RC * NPL // NTEC)],
                        out_hbm.at[cid, pl.ds((lo + sid * (RC // NTEC)) * NPL,
                                              RC * NPL // NTEC)])
        plsc.subcore_barrier()
        return carry

    jax.lax.fori_loop(0, NCHUNK, chunk_body, 0)


def _edge_pass(xc, src, dst, emb, zrows):
    mesh = plsc.VectorSubcoreMesh(core_axis_name="c", subcore_axis_name="s",
                                  num_cores=NSC, num_subcores=NTEC)
    f = pl.kernel(
        _edge_body,
        out_type=jax.ShapeDtypeStruct((NSC, NP * NPL, 128), jnp.float32),
        mesh=mesh,
        compiler_params=pltpu.CompilerParams(needs_layout_passes=False),
        scratch_types=[
            pltpu.VMEM((EPT + 8,), jnp.int32),
            pltpu.VMEM((EPT + 8,), jnp.int32),
            pltpu.VMEM((HITCAP,), jnp.int32),
            pltpu.VMEM((HITCAP,), jnp.int32),
            pltpu.VMEM((HITCAP,), jnp.int32),
            pltpu.VMEM((16 * NPL,), jnp.int32),
            pltpu.VMEM((16 * NPL,), jnp.int32),
            pltpu.VMEM((16 * NPL,), jnp.int32),
            pltpu.VMEM((16 * NPL,), jnp.int32),
            pltpu.VMEM((16 * NPL,), jnp.int32),
            pltpu.VMEM((16 * NPL,), jnp.int32),
            pltpu.VMEM((16 * NPL, 128), jnp.float32),
            pltpu.VMEM((16 * NPL, 128), jnp.float32),
            pltpu.VMEM((16 * NPL, 128), jnp.float32),
            pltpu.VMEM((16 * NPL, 128), jnp.float32),
            pltpu.VMEM_SHARED((RCA * NPL, 128), jnp.float32),
            pltpu.SemaphoreType.DMA,
            pltpu.SemaphoreType.DMA,
            pltpu.SemaphoreType.DMA,
        ],
    )
    return f(xc, src, dst, emb, zrows)


def _node_body(xc_ref, ag_ref, wn_ref, bn_ref, g_ref, bt_ref, out_ref):
    t = jnp.concatenate(
        [xc_ref[:, k, :] + ag_ref[0, :, k, :] + ag_ref[1, :, k, :]
         for k in range(NPL)], axis=1)
    o = jnp.dot(t, wn_ref[...], preferred_element_type=jnp.float32) + bn_ref[...]
    o = o * (_INV_BN) * g_ref[...] + bt_ref[...]
    o = 0.5 * o * (1.0 + jax.lax.erf(o * _INV_SQRT2))
    out_ref[:, 0, :] = xc_ref[:, 0, :] + o
    for k in range(1, NPL):
        out_ref[:, k, :] = xc_ref[:, k, :]


def _node_update(xc, aggr, Wn_pad, bn, g, bt):
    return pl.pallas_call(
        _node_body,
        grid=(NGRID,),
        in_specs=[
            pl.BlockSpec((NB, NPL, 128), lambda i: (i, 0, 0)),
            pl.BlockSpec((NSC, NB, NPL, 128), lambda i: (0, i, 0, 0)),
            pl.BlockSpec((IN_PAD, HID), lambda i: (0, 0)),
            pl.BlockSpec((HID,), lambda i: (0,)),
            pl.BlockSpec((HID,), lambda i: (0,)),
            pl.BlockSpec((HID,), lambda i: (0,)),
        ],
        out_specs=pl.BlockSpec((NB, NPL, 128), lambda i: (i, 0, 0)),
        out_shape=jax.ShapeDtypeStruct((NP, NPL, 128), jnp.float32),
    )(xc, aggr, Wn_pad, bn, g, bt)


def _final_body(h_ref, batch_ref, wl_ref, bl_ref, out_ref):
    i = pl.program_id(0)
    hw = jnp.dot(h_ref[:, 0, :], wl_ref[...], preferred_element_type=jnp.float32)
    hw = hw + bl_ref[...]
    b = batch_ref[0, 0, :]
    oh = (b[:, None] == jax.lax.broadcasted_iota(jnp.int32, (NB, B), 1)).astype(jnp.float32)
    contrib = jnp.dot(oh.T, hw, preferred_element_type=jnp.float32)

    @pl.when(i == 0)
    def _():
        out_ref[...] = contrib

    @pl.when(i != 0)
    def _():
        out_ref[...] += contrib


def _final_pool(xc, batch2, Wl, bl):
    return pl.pallas_call(
        _final_body,
        grid=(NGRID,),
        in_specs=[
            pl.BlockSpec((NB, NPL, HID), lambda i: (i, 0, 0)),
            pl.BlockSpec((1, 1, NB), lambda i: (i, 0, 0)),
            pl.BlockSpec((HID, OUT), lambda i: (0, 0)),
            pl.BlockSpec((OUT,), lambda i: (0,)),
        ],
        out_specs=pl.BlockSpec((B, OUT), lambda i: (0, 0)),
        out_shape=jax.ShapeDtypeStruct((B, OUT), jnp.float32),
    )(xc, batch2, Wl, bl)


def kernel(x, pos, edge_index, edge_attr, batch, context_vector,
           W0, b0,
           Wn0, bn0, We0, be0, g0, bt0,
           Wn1, bn1, We1, be1, g1, bt1,
           Wn2, bn2, We2, be2, g2, bt2,
           Wl, bl):
    x2 = x.reshape(NGRID, 1, NB).astype(jnp.int32)
    pos3 = pos.reshape(NGRID, NB, 3)
    batch2 = batch.reshape(NGRID, 1, NB).astype(jnp.int32)
    src = edge_index[0].astype(jnp.int32)
    dst = edge_index[1].astype(jnp.int32)
    zrows = jnp.zeros((RCA * NPL // NTEC, 128), jnp.float32)

    xc = _prep(x2, pos3, batch2, context_vector, W0, b0)

    layers = [(Wn0, bn0, We0, be0, g0, bt0),
              (Wn1, bn1, We1, be1, g1, bt1),
              (Wn2, bn2, We2, be2, g2, bt2)]
    for (Wn, bn, We, be, g, bt) in layers:
        We_pad = jnp.pad(We, ((0, 0), (0, IN_PAD - IN_DIM)))
        be_pad = jnp.pad(be, ((0, IN_PAD - IN_DIM),))
        Wn_pad = jnp.pad(Wn, ((0, IN_PAD - IN_DIM), (0, 0)))
        emb = _emb(edge_attr, We_pad, be_pad)
        aggr = _edge_pass(xc.reshape(NP * NPL, 128), src, dst,
                          emb.reshape(E * NPL, 128), zrows)
        xc = _node_update(xc, aggr.reshape(NSC, NP, NPL, 128), Wn_pad, bn, g, bt)

    return _final_pool(xc, batch2, Wl, bl)


# speedup vs baseline: 1.2569x; 1.2569x over previous
import jax
import jax.numpy as jnp
import numpy as np
from jax.experimental import pallas as pl
from jax.experimental.pallas import tpu as pltpu
from jax.experimental.pallas import tpu_sc as plsc

N = 10000
E = 160000
B = 64
CTX = 512
PED = 240
HID = 128
IN_DIM = HID + PED + CTX
IN_PAD = 896
OUT = 1024

NB = 1000
NGRID = N // NB

NSC = 2
NTEC = 16
EPT = E // (NSC * NTEC)
RC = 512
NCHUNK = 21
NP = NCHUNK * RC
RCA = 640
G = 32
NSCAN = (EPT + 15) // 16
HITCAP = 5120
NPL = IN_PAD // 128

_CD = PED // 3
_INV_BN = float(1.0 / np.sqrt(1.0 + 1e-5))
_INV_SQRT2 = float(1.0 / np.sqrt(2.0))


def _prep_body(x_ref, pos_ref, batch_ref, ctx_ref, w0_ref, b0_ref, out_ref):
    xb = x_ref[0, 0, :]
    oh = (xb[:, None] == jax.lax.broadcasted_iota(jnp.int32, (NB, 118), 1)).astype(jnp.float32)
    h0 = jnp.dot(oh, w0_ref[...], preferred_element_type=jnp.float32) + b0_ref[...]
    pb = pos_ref[0]
    dt = jnp.exp(jax.lax.broadcasted_iota(jnp.int32, (1, _CD // 2), 1).astype(jnp.float32)
                 * (-(np.log(10000.0) / (PED // 2))))
    parts = []
    for i in range(3):
        s = pb[:, i:i + 1] * dt
        parts.append(jnp.concatenate([jnp.sin(s), jnp.cos(s)], axis=-1))
    pe = jnp.concatenate(parts, axis=1)
    bb = batch_ref[0, 0, :]
    ohb = (bb[:, None] == jax.lax.broadcasted_iota(jnp.int32, (NB, B), 1)).astype(jnp.float32)
    ctxg = jnp.dot(ohb, ctx_ref[...], preferred_element_type=jnp.float32)
    pad = jnp.zeros((NB, IN_PAD - IN_DIM), dtype=jnp.float32)
    xcb = jnp.concatenate([h0, pe, ctxg, pad], axis=1)
    for k in range(NPL):
        out_ref[k] = xcb[:, k * 128:(k + 1) * 128]


def _prep(x2, pos3, batch2, ctx, W0, b0):
    return pl.pallas_call(
        _prep_body,
        grid=(NGRID,),
        in_specs=[
            pl.BlockSpec((1, 1, NB), lambda i: (i, 0, 0)),
            pl.BlockSpec((1, NB, 3), lambda i: (i, 0, 0)),
            pl.BlockSpec((1, 1, NB), lambda i: (i, 0, 0)),
            pl.BlockSpec((B, CTX), lambda i: (0, 0)),
            pl.BlockSpec((118, HID), lambda i: (0, 0)),
            pl.BlockSpec((HID,), lambda i: (0,)),
        ],
        out_specs=pl.BlockSpec((NPL, NB, 128), lambda i: (0, i, 0)),
        out_shape=jax.ShapeDtypeStruct((NPL, NP, 128), jnp.float32),
    )(x2, pos3, batch2, ctx, W0, b0)


EB = 2000


def _emb_body(ea_ref, we_ref, be_ref, out_ref):
    emb = jnp.dot(ea_ref[...], we_ref[...], preferred_element_type=jnp.float32) + be_ref[...]
    for k in range(NPL):
        out_ref[k] = emb[:, k * 128:(k + 1) * 128]


def _emb(edge_attr, We_pad, be_pad):
    return pl.pallas_call(
        _emb_body,
        grid=(E // EB,),
        in_specs=[
            pl.BlockSpec((EB, 5), lambda i: (i, 0)),
            pl.BlockSpec((5, IN_PAD), lambda i: (0, 0)),
            pl.BlockSpec((IN_PAD,), lambda i: (0,)),
        ],
        out_specs=pl.BlockSpec((NPL, EB, 128), lambda i: (0, i, 0)),
        out_shape=jax.ShapeDtypeStruct((NPL, E, 128), jnp.float32),
    )(edge_attr, We_pad, be_pad)


def _edge_body(xc_hbm, src_hbm, dst_hbm, emb_hbm, z_hbm,
               out_hbm,
               src_v, dst_v, srch, dsth, ehh,
               dstg0, dstg1, erows, rows, acc, sem, sem2, sem3):
    cid = jax.lax.axis_index("c")
    sid = jax.lax.axis_index("s")
    wid = cid * NTEC + sid
    base = wid * EPT
    pltpu.sync_copy(src_hbm.at[pl.ds(base, EPT)], src_v.at[pl.ds(0, EPT)])
    pltpu.sync_copy(dst_hbm.at[pl.ds(base, EPT)], dst_v.at[pl.ds(0, EPT)])
    lanes = jax.lax.iota(jnp.int32, 16)
    full = lanes >= 0

    def chunk_body(c, carry):
        lo = c * RC
        for k in range(NPL):
            pltpu.sync_copy(z_hbm, acc.at[k, pl.ds(sid * (RCA // NTEC), RCA // NTEC)])
        plsc.subcore_barrier()

        def scan_body(i, ptr):
            d = dst_v[pl.ds(i * 16, 16)]
            s = src_v[pl.ds(i * 16, 16)]
            eidx = i * 16 + lanes
            m = (d >= lo) & (d < lo + RC) & (eidx < EPT)
            mi = m.astype(jnp.int32)
            pos = ptr + plsc.cumsum(mi) - 1
            plsc.store_scatter(srch, [pos], s, mask=m)
            plsc.store_scatter(dsth, [pos], d - lo, mask=m)
            plsc.store_scatter(ehh, [pos], base + eidx, mask=m)
            return ptr + jnp.sum(mi)

        nh = jax.lax.fori_loop(0, NSCAN, scan_body, jnp.int32(0))
        zeros16 = jnp.zeros((16,), jnp.int32)
        for off in (0, 16):
            pos = nh + off + lanes
            plsc.store_scatter(srch, [pos], zeros16, mask=full)
            plsc.store_scatter(dsth, [pos], zeros16 + RC, mask=full)
            plsc.store_scatter(ehh, [pos], zeros16, mask=full)
        ng = (nh + (G - 1)) // G

        def group_body(g, _):
            gb = g * G
            gcop = []
            for h in range(2):
                idx = srch.at[pl.ds(gb + h * 16, 16)]
                eidxr = ehh.at[pl.ds(gb + h * 16, 16)]
                gcop.append(
                    [pltpu.async_copy(xc_hbm.at[k].at[idx], rows.at[h, k], sem)
                     for k in range(NPL)]
                    + [pltpu.async_copy(emb_hbm.at[k].at[eidxr], erows.at[h, k], sem2)
                       for k in range(NPL)])

            scat = []
            dstgs = (dstg0, dstg1)
            for h in range(2):
                for cpy in gcop[h]:
                    cpy.wait()

                def e_body(e, _, h=h):
                    for k in range(NPL):
                        for jj in range(8):
                            cs = pl.ds(jj * 16, 16)
                            v = rows[h, k, e, cs] + erows[h, k, e, cs]
                            rows[h, k, e, cs] = jnp.maximum(v, 0.0)
                    return 0

                jax.lax.fori_loop(0, 16, e_body, 0)
                dstgs[h][pl.ds(0, 16)] = dsth[pl.ds(gb + h * 16, 16)]
                for k in range(NPL):
                    scat.append(pltpu.async_copy(rows.at[h, k], acc.at[k].at[dstgs[h]],
                                                 sem3, add=True))
            for s_ in scat:
                s_.wait()
            return 0

        jax.lax.fori_loop(0, ng, group_body, 0)
        plsc.subcore_barrier()
        for k in range(NPL):
            pltpu.sync_copy(acc.at[k, pl.ds(sid * (RC // NTEC), RC // NTEC)],
                            out_hbm.at[cid, k, pl.ds(lo + sid * (RC // NTEC), RC // NTEC), :])
        plsc.subcore_barrier()
        return carry

    jax.lax.fori_loop(0, NCHUNK, chunk_body, 0)


def _edge_pass(xc, src, dst, emb, zrows):
    mesh = plsc.VectorSubcoreMesh(core_axis_name="c", subcore_axis_name="s",
                                  num_cores=NSC, num_subcores=NTEC)
    f = pl.kernel(
        _edge_body,
        out_type=jax.ShapeDtypeStruct((NSC, NPL, NP, 128), jnp.float32),
        mesh=mesh,
        compiler_params=pltpu.CompilerParams(needs_layout_passes=False),
        scratch_types=[
            pltpu.VMEM((EPT + 8,), jnp.int32),
            pltpu.VMEM((EPT + 8,), jnp.int32),
            pltpu.VMEM((HITCAP,), jnp.int32),
            pltpu.VMEM((HITCAP,), jnp.int32),
            pltpu.VMEM((HITCAP,), jnp.int32),
            pltpu.VMEM((16,), jnp.int32),
            pltpu.VMEM((16,), jnp.int32),
            pltpu.VMEM((2, NPL, 16, 128), jnp.float32),
            pltpu.VMEM((2, NPL, 16, 128), jnp.float32),
            pltpu.VMEM_SHARED((NPL, RCA, 128), jnp.float32),
            pltpu.SemaphoreType.DMA,
            pltpu.SemaphoreType.DMA,
            pltpu.SemaphoreType.DMA,
        ],
    )
    return f(xc, src, dst, emb, zrows)


def _node_body(xc_ref, ag_ref, wn_ref, bn_ref, g_ref, bt_ref, out_ref):
    t = jnp.concatenate(
        [xc_ref[k] + ag_ref[0, k] + ag_ref[1, k] for k in range(NPL)], axis=1)
    o = jnp.dot(t, wn_ref[...], preferred_element_type=jnp.float32) + bn_ref[...]
    o = o * (_INV_BN) * g_ref[...] + bt_ref[...]
    o = 0.5 * o * (1.0 + jax.lax.erf(o * _INV_SQRT2))
    out_ref[0] = xc_ref[0] + o
    for k in range(1, NPL):
        out_ref[k] = xc_ref[k]


def _node_update(xc, aggr, Wn_pad, bn, g, bt):
    return pl.pallas_call(
        _node_body,
        grid=(NGRID,),
        in_specs=[
            pl.BlockSpec((NPL, NB, 128), lambda i: (0, i, 0)),
            pl.BlockSpec((NSC, NPL, NB, 128), lambda i: (0, 0, i, 0)),
            pl.BlockSpec((IN_PAD, HID), lambda i: (0, 0)),
            pl.BlockSpec((HID,), lambda i: (0,)),
            pl.BlockSpec((HID,), lambda i: (0,)),
            pl.BlockSpec((HID,), lambda i: (0,)),
        ],
        out_specs=pl.BlockSpec((NPL, NB, 128), lambda i: (0, i, 0)),
        out_shape=jax.ShapeDtypeStruct((NPL, NP, 128), jnp.float32),
    )(xc, aggr, Wn_pad, bn, g, bt)


def _final_body(h_ref, batch_ref, wl_ref, bl_ref, out_ref):
    i = pl.program_id(0)
    hw = jnp.dot(h_ref[0], wl_ref[...], preferred_element_type=jnp.float32)
    hw = hw + bl_ref[...]
    b = batch_ref[0, 0, :]
    oh = (b[:, None] == jax.lax.broadcasted_iota(jnp.int32, (NB, B), 1)).astype(jnp.float32)
    contrib = jnp.dot(oh.T, hw, preferred_element_type=jnp.float32)

    @pl.when(i == 0)
    def _():
        out_ref[...] = contrib

    @pl.when(i != 0)
    def _():
        out_ref[...] += contrib


def _final_pool(xc, batch2, Wl, bl):
    return pl.pallas_call(
        _final_body,
        grid=(NGRID,),
        in_specs=[
            pl.BlockSpec((1, NB, HID), lambda i: (0, i, 0)),
            pl.BlockSpec((1, 1, NB), lambda i: (i, 0, 0)),
            pl.BlockSpec((HID, OUT), lambda i: (0, 0)),
            pl.BlockSpec((OUT,), lambda i: (0,)),
        ],
        out_specs=pl.BlockSpec((B, OUT), lambda i: (0, 0)),
        out_shape=jax.ShapeDtypeStruct((B, OUT), jnp.float32),
    )(xc, batch2, Wl, bl)


def kernel(x, pos, edge_index, edge_attr, batch, context_vector,
           W0, b0,
           Wn0, bn0, We0, be0, g0, bt0,
           Wn1, bn1, We1, be1, g1, bt1,
           Wn2, bn2, We2, be2, g2, bt2,
           Wl, bl):
    x2 = x.reshape(NGRID, 1, NB).astype(jnp.int32)
    pos3 = pos.reshape(NGRID, NB, 3)
    batch2 = batch.reshape(NGRID, 1, NB).astype(jnp.int32)
    src = edge_index[0].astype(jnp.int32)
    dst = edge_index[1].astype(jnp.int32)
    zrows = jnp.zeros((RCA // NTEC, 128), jnp.float32)

    xc = _prep(x2, pos3, batch2, context_vector, W0, b0)

    layers = [(Wn0, bn0, We0, be0, g0, bt0),
              (Wn1, bn1, We1, be1, g1, bt1),
              (Wn2, bn2, We2, be2, g2, bt2)]
    for (Wn, bn, We, be, g, bt) in layers:
        We_pad = jnp.pad(We, ((0, 0), (0, IN_PAD - IN_DIM)))
        be_pad = jnp.pad(be, ((0, IN_PAD - IN_DIM),))
        Wn_pad = jnp.pad(Wn, ((0, IN_PAD - IN_DIM), (0, 0)))
        emb = _emb(edge_attr, We_pad, be_pad)
        aggr = _edge_pass(xc, src, dst, emb, zrows)
        xc = _node_update(xc, aggr, Wn_pad, bn, g, bt)

    return _final_pool(xc, batch2, Wl, bl)


# speedup vs baseline: 1.2896x; 1.0260x over previous
import jax
import jax.numpy as jnp
import numpy as np
from jax.experimental import pallas as pl
from jax.experimental.pallas import tpu as pltpu
from jax.experimental.pallas import tpu_sc as plsc

N = 10000
E = 160000
B = 64
CTX = 512
PED = 240
HID = 128
IN_DIM = HID + PED + CTX
IN_PAD = 896
OUT = 1024

NB = 1000
NGRID = N // NB

NSC = 2
NTEC = 16
EPT = E // (NSC * NTEC)
RC = 640
NCHUNK = 16
NP = NCHUNK * RC
RCA = 768
G = 32
NSCAN = (EPT + 15) // 16
HITCAP = 5120
NPL = IN_PAD // 128

_CD = PED // 3
_INV_BN = float(1.0 / np.sqrt(1.0 + 1e-5))
_INV_SQRT2 = float(1.0 / np.sqrt(2.0))


def _prep_body(x_ref, pos_ref, batch_ref, ctx_ref, w0_ref, b0_ref, out_ref):
    xb = x_ref[0, 0, :]
    oh = (xb[:, None] == jax.lax.broadcasted_iota(jnp.int32, (NB, 118), 1)).astype(jnp.float32)
    h0 = jnp.dot(oh, w0_ref[...], preferred_element_type=jnp.float32) + b0_ref[...]
    pb = pos_ref[0]
    dt = jnp.exp(jax.lax.broadcasted_iota(jnp.int32, (1, _CD // 2), 1).astype(jnp.float32)
                 * (-(np.log(10000.0) / (PED // 2))))
    parts = []
    for i in range(3):
        s = pb[:, i:i + 1] * dt
        parts.append(jnp.concatenate([jnp.sin(s), jnp.cos(s)], axis=-1))
    pe = jnp.concatenate(parts, axis=1)
    bb = batch_ref[0, 0, :]
    ohb = (bb[:, None] == jax.lax.broadcasted_iota(jnp.int32, (NB, B), 1)).astype(jnp.float32)
    ctxg = jnp.dot(ohb, ctx_ref[...], preferred_element_type=jnp.float32)
    pad = jnp.zeros((NB, IN_PAD - IN_DIM), dtype=jnp.float32)
    xcb = jnp.concatenate([h0, pe, ctxg, pad], axis=1)
    for k in range(NPL):
        out_ref[k] = xcb[:, k * 128:(k + 1) * 128]


def _prep(x2, pos3, batch2, ctx, W0, b0):
    return pl.pallas_call(
        _prep_body,
        grid=(NGRID,),
        in_specs=[
            pl.BlockSpec((1, 1, NB), lambda i: (i, 0, 0)),
            pl.BlockSpec((1, NB, 3), lambda i: (i, 0, 0)),
            pl.BlockSpec((1, 1, NB), lambda i: (i, 0, 0)),
            pl.BlockSpec((B, CTX), lambda i: (0, 0)),
            pl.BlockSpec((118, HID), lambda i: (0, 0)),
            pl.BlockSpec((HID,), lambda i: (0,)),
        ],
        out_specs=pl.BlockSpec((NPL, NB, 128), lambda i: (0, i, 0)),
        out_shape=jax.ShapeDtypeStruct((NPL, NP, 128), jnp.float32),
    )(x2, pos3, batch2, ctx, W0, b0)


EB = 2000


def _emb_body(ea_ref, we_ref, be_ref, out_ref):
    emb = jnp.dot(ea_ref[...], we_ref[...], preferred_element_type=jnp.float32) + be_ref[...]
    for k in range(NPL):
        out_ref[k] = emb[:, k * 128:(k + 1) * 128]


def _emb(edge_attr, We_pad, be_pad):
    return pl.pallas_call(
        _emb_body,
        grid=(E // EB,),
        in_specs=[
            pl.BlockSpec((EB, 5), lambda i: (i, 0)),
            pl.BlockSpec((5, IN_PAD), lambda i: (0, 0)),
            pl.BlockSpec((IN_PAD,), lambda i: (0,)),
        ],
        out_specs=pl.BlockSpec((NPL, EB, 128), lambda i: (0, i, 0)),
        out_shape=jax.ShapeDtypeStruct((NPL, E, 128), jnp.float32),
    )(edge_attr, We_pad, be_pad)


def _edge_body(xc_hbm, src_hbm, dst_hbm, emb_hbm, z_hbm,
               out_hbm,
               src_v, dst_v, srch, dsth, ehh,
               dstg0, dstg1, erows, rows, acc, sem, sem2, sem3):
    cid = jax.lax.axis_index("c")
    sid = jax.lax.axis_index("s")
    wid = cid * NTEC + sid
    base = wid * EPT
    pltpu.sync_copy(src_hbm.at[pl.ds(base, EPT)], src_v.at[pl.ds(0, EPT)])
    pltpu.sync_copy(dst_hbm.at[pl.ds(base, EPT)], dst_v.at[pl.ds(0, EPT)])
    lanes = jax.lax.iota(jnp.int32, 16)
    full = lanes >= 0

    def chunk_body(c, carry):
        lo = c * RC
        for k in range(NPL):
            pltpu.sync_copy(z_hbm, acc.at[k, pl.ds(sid * (RCA // NTEC), RCA // NTEC)])
        plsc.subcore_barrier()

        def scan_body(i, ptr):
            d = dst_v[pl.ds(i * 16, 16)]
            s = src_v[pl.ds(i * 16, 16)]
            eidx = i * 16 + lanes
            m = (d >= lo) & (d < lo + RC) & (eidx < EPT)
            mi = m.astype(jnp.int32)
            pos = ptr + plsc.cumsum(mi) - 1
            plsc.store_scatter(srch, [pos], s, mask=m)
            plsc.store_scatter(dsth, [pos], d - lo, mask=m)
            plsc.store_scatter(ehh, [pos], base + eidx, mask=m)
            return ptr + jnp.sum(mi)

        nh = jax.lax.fori_loop(0, NSCAN, scan_body, jnp.int32(0))
        zeros16 = jnp.zeros((16,), jnp.int32)
        for off in (0, 16):
            pos = nh + off + lanes
            plsc.store_scatter(srch, [pos], zeros16, mask=full)
            plsc.store_scatter(dsth, [pos], zeros16 + RC, mask=full)
            plsc.store_scatter(ehh, [pos], zeros16, mask=full)
        ng = (nh + (G - 1)) // G

        def group_body(g, _):
            gb = g * G
            gcop = []
            for h in range(2):
                idx = srch.at[pl.ds(gb + h * 16, 16)]
                eidxr = ehh.at[pl.ds(gb + h * 16, 16)]
                gcop.append(
                    [pltpu.async_copy(xc_hbm.at[k].at[idx], rows.at[h, k], sem)
                     for k in range(NPL)]
                    + [pltpu.async_copy(emb_hbm.at[k].at[eidxr], erows.at[h, k], sem2)
                       for k in range(NPL)])

            scat = []
            dstgs = (dstg0, dstg1)
            for h in range(2):
                for cpy in gcop[h]:
                    cpy.wait()

                def e_body(e, _, h=h):
                    for k in range(NPL):
                        for jj in range(8):
                            cs = pl.ds(jj * 16, 16)
                            v = rows[h, k, e, cs] + erows[h, k, e, cs]
                            rows[h, k, e, cs] = jnp.maximum(v, 0.0)
                    return 0

                jax.lax.fori_loop(0, 16, e_body, 0)
                dstgs[h][pl.ds(0, 16)] = dsth[pl.ds(gb + h * 16, 16)]
                for k in range(NPL):
                    scat.append(pltpu.async_copy(rows.at[h, k], acc.at[k].at[dstgs[h]],
                                                 sem3, add=True))
            for s_ in scat:
                s_.wait()
            return 0

        jax.lax.fori_loop(0, ng, group_body, 0)
        plsc.subcore_barrier()
        for k in range(NPL):
            pltpu.sync_copy(acc.at[k, pl.ds(sid * (RC // NTEC), RC // NTEC)],
                            out_hbm.at[cid, k, pl.ds(lo + sid * (RC // NTEC), RC // NTEC), :])
        plsc.subcore_barrier()
        return carry

    jax.lax.fori_loop(0, NCHUNK, chunk_body, 0)


def _edge_pass(xc, src, dst, emb, zrows):
    mesh = plsc.VectorSubcoreMesh(core_axis_name="c", subcore_axis_name="s",
                                  num_cores=NSC, num_subcores=NTEC)
    f = pl.kernel(
        _edge_body,
        out_type=jax.ShapeDtypeStruct((NSC, NPL, NP, 128), jnp.float32),
        mesh=mesh,
        compiler_params=pltpu.CompilerParams(needs_layout_passes=False),
        scratch_types=[
            pltpu.VMEM((EPT + 8,), jnp.int32),
            pltpu.VMEM((EPT + 8,), jnp.int32),
            pltpu.VMEM((HITCAP,), jnp.int32),
            pltpu.VMEM((HITCAP,), jnp.int32),
            pltpu.VMEM((HITCAP,), jnp.int32),
            pltpu.VMEM((16,), jnp.int32),
            pltpu.VMEM((16,), jnp.int32),
            pltpu.VMEM((2, NPL, 16, 128), jnp.float32),
            pltpu.VMEM((2, NPL, 16, 128), jnp.float32),
            pltpu.VMEM_SHARED((NPL, RCA, 128), jnp.float32),
            pltpu.SemaphoreType.DMA,
            pltpu.SemaphoreType.DMA,
            pltpu.SemaphoreType.DMA,
        ],
    )
    return f(xc, src, dst, emb, zrows)


def _node_body(xc_ref, ag_ref, wn_ref, bn_ref, g_ref, bt_ref, out_ref):
    t = jnp.concatenate(
        [xc_ref[k] + ag_ref[0, k] + ag_ref[1, k] for k in range(NPL)], axis=1)
    o = jnp.dot(t, wn_ref[...], preferred_element_type=jnp.float32) + bn_ref[...]
    o = o * (_INV_BN) * g_ref[...] + bt_ref[...]
    o = 0.5 * o * (1.0 + jax.lax.erf(o * _INV_SQRT2))
    out_ref[0] = xc_ref[0] + o
    for k in range(1, NPL):
        out_ref[k] = xc_ref[k]


def _node_update(xc, aggr, Wn_pad, bn, g, bt):
    return pl.pallas_call(
        _node_body,
        grid=(NGRID,),
        in_specs=[
            pl.BlockSpec((NPL, NB, 128), lambda i: (0, i, 0)),
            pl.BlockSpec((NSC, NPL, NB, 128), lambda i: (0, 0, i, 0)),
            pl.BlockSpec((IN_PAD, HID), lambda i: (0, 0)),
            pl.BlockSpec((HID,), lambda i: (0,)),
            pl.BlockSpec((HID,), lambda i: (0,)),
            pl.BlockSpec((HID,), lambda i: (0,)),
        ],
        out_specs=pl.BlockSpec((NPL, NB, 128), lambda i: (0, i, 0)),
        out_shape=jax.ShapeDtypeStruct((NPL, NP, 128), jnp.float32),
    )(xc, aggr, Wn_pad, bn, g, bt)


def _final_body(h_ref, batch_ref, wl_ref, bl_ref, out_ref):
    i = pl.program_id(0)
    hw = jnp.dot(h_ref[0], wl_ref[...], preferred_element_type=jnp.float32)
    hw = hw + bl_ref[...]
    b = batch_ref[0, 0, :]
    oh = (b[:, None] == jax.lax.broadcasted_iota(jnp.int32, (NB, B), 1)).astype(jnp.float32)
    contrib = jnp.dot(oh.T, hw, preferred_element_type=jnp.float32)

    @pl.when(i == 0)
    def _():
        out_ref[...] = contrib

    @pl.when(i != 0)
    def _():
        out_ref[...] += contrib


def _final_pool(xc, batch2, Wl, bl):
    return pl.pallas_call(
        _final_body,
        grid=(NGRID,),
        in_specs=[
            pl.BlockSpec((1, NB, HID), lambda i: (0, i, 0)),
            pl.BlockSpec((1, 1, NB), lambda i: (i, 0, 0)),
            pl.BlockSpec((HID, OUT), lambda i: (0, 0)),
            pl.BlockSpec((OUT,), lambda i: (0,)),
        ],
        out_specs=pl.BlockSpec((B, OUT), lambda i: (0, 0)),
        out_shape=jax.ShapeDtypeStruct((B, OUT), jnp.float32),
    )(xc, batch2, Wl, bl)


def kernel(x, pos, edge_index, edge_attr, batch, context_vector,
           W0, b0,
           Wn0, bn0, We0, be0, g0, bt0,
           Wn1, bn1, We1, be1, g1, bt1,
           Wn2, bn2, We2, be2, g2, bt2,
           Wl, bl):
    x2 = x.reshape(NGRID, 1, NB).astype(jnp.int32)
    pos3 = pos.reshape(NGRID, NB, 3)
    batch2 = batch.reshape(NGRID, 1, NB).astype(jnp.int32)
    src = edge_index[0].astype(jnp.int32)
    dst = edge_index[1].astype(jnp.int32)
    zrows = jnp.zeros((RCA // NTEC, 128), jnp.float32)

    xc = _prep(x2, pos3, batch2, context_vector, W0, b0)

    layers = [(Wn0, bn0, We0, be0, g0, bt0),
              (Wn1, bn1, We1, be1, g1, bt1),
              (Wn2, bn2, We2, be2, g2, bt2)]
    for (Wn, bn, We, be, g, bt) in layers:
        We_pad = jnp.pad(We, ((0, 0), (0, IN_PAD - IN_DIM)))
        be_pad = jnp.pad(be, ((0, IN_PAD - IN_DIM),))
        Wn_pad = jnp.pad(Wn, ((0, IN_PAD - IN_DIM), (0, 0)))
        emb = _emb(edge_attr, We_pad, be_pad)
        aggr = _edge_pass(xc, src, dst, emb, zrows)
        xc = _node_update(xc, aggr, Wn_pad, bn, g, bt)

    return _final_pool(xc, batch2, Wl, bl)
